# Initial kernel scaffold; baseline (speedup 1.0000x reference)
import jax
import jax.numpy as jnp
from jax.experimental import pallas as pl


def kernel(pred):
    raise NotImplementedError("write your pallas kernel here")


# speedup vs baseline: 8.8135x; 8.8135x over previous
import jax
import jax.numpy as jnp
from jax.experimental import pallas as pl
from jax.experimental.pallas import tpu as pltpu

_CONF_THRESH = 0.25
_IOU_THRESH = 0.45
_MAX_DET = 100
_NC = 80
_ROWS = 160
_LANES = 128
_NPAD = _ROWS * _LANES


def _nms_body(p_ref, o_ref, ext_ref):
    p = p_ref[0]
    obj = p[4]

    mc = p[5] * obj
    for c in range(1, _NC):
        mc = jnp.maximum(mc, p[5 + c] * obj)
    carg = jnp.zeros((_ROWS, _LANES), jnp.int32)
    for c in range(_NC - 1, -1, -1):
        carg = jnp.where(p[5 + c] * obj == mc, c, carg)

    scores = jnp.where(mc > _CONF_THRESH, mc, -1.0)

    x = p[0]
    y = p[1]
    w = p[2]
    h = p[3]
    x1 = x - w / 2.0
    y1 = y - h / 2.0
    x2 = x + w / 2.0
    y2 = y + h / 2.0
    clsf = carg.astype(jnp.float32)
    off = clsf * 4096.0
    bx1 = x1 + off
    by1 = y1 + off
    bx2 = x2 + off
    by2 = y2 + off
    a2 = (bx2 - bx1) * (by2 - by1)

    ext_ref[0] = x1
    ext_ref[1] = y1
    ext_ref[2] = x2
    ext_ref[3] = y2
    ext_ref[4] = clsf

    riota = jax.lax.broadcasted_iota(jnp.int32, (_ROWS, _LANES), 0)
    liota = jax.lax.broadcasted_iota(jnp.int32, (_ROWS, _LANES), 1)
    fiota = riota * _LANES + liota
    lane1 = jax.lax.broadcasted_iota(jnp.int32, (1, _LANES), 1)

    def body(i, sc):
        m = jnp.max(sc)
        idx = jnp.min(jnp.where(sc == m, fiota, _NPAD))
        r = idx // _LANES
        cc = idx - r * _LANES

        def ex(k):
            vrow = ext_ref[k, pl.ds(r, 1), :]
            return jnp.sum(jnp.where(lane1 == cc, vrow, 0.0))

        ex1 = ex(0)
        ey1 = ex(1)
        ex2_ = ex(2)
        ey2 = ex(3)
        ecls = ex(4)
        eoff = ecls * 4096.0
        ebx1 = ex1 + eoff
        eby1 = ey1 + eoff
        ebx2 = ex2_ + eoff
        eby2 = ey2 + eoff

        xx1 = jnp.maximum(ebx1, bx1)
        yy1 = jnp.maximum(eby1, by1)
        xx2 = jnp.minimum(ebx2, bx2)
        yy2 = jnp.minimum(eby2, by2)
        inter = jnp.maximum(xx2 - xx1, 0.0) * jnp.maximum(yy2 - yy1, 0.0)
        a1 = (ebx2 - ebx1) * (eby2 - eby1)
        iou = inter / (a1 + a2 - inter + 1e-9)

        valid = m > 0.0
        sup = (iou > _IOU_THRESH) & valid
        sc2 = jnp.where(sup | (fiota == idx), -1.0, sc)

        vf = jnp.where(valid, 1.0, 0.0)
        conf = jnp.maximum(m, 0.0)
        row = (
            jnp.where(lane1 == 0, ex1, 0.0)
            + jnp.where(lane1 == 1, ey1, 0.0)
            + jnp.where(lane1 == 2, ex2_, 0.0)
            + jnp.where(lane1 == 3, ey2, 0.0)
            + jnp.where(lane1 == 4, conf, 0.0)
            + jnp.where(lane1 == 5, ecls, 0.0)
        ) * vf
        o_ref[0, pl.ds(i, 1), :] = row
        return sc2

    jax.lax.fori_loop(0, _MAX_DET, body, scores)


def _run(predT, interpret=False):
    b, c = predT.shape[0], predT.shape[1]
    return pl.pallas_call(
        _nms_body,
        grid=(b,),
        in_specs=[pl.BlockSpec((1, c, _ROWS, _LANES), lambda i: (i, 0, 0, 0))],
        out_specs=pl.BlockSpec((1, 128, _LANES), lambda i: (i, 0, 0)),
        out_shape=jax.ShapeDtypeStruct((b, 128, _LANES), jnp.float32),
        scratch_shapes=[pltpu.VMEM((5, _ROWS, _LANES), jnp.float32)],
        compiler_params=pltpu.CompilerParams(
            dimension_semantics=("arbitrary",)
        ),
        interpret=interpret,
    )(predT)


def kernel(pred):
    b, n, c = pred.shape
    predT = jnp.swapaxes(pred, 1, 2)
    predT = jnp.pad(predT, ((0, 0), (0, 0), (0, _NPAD - n)))
    predT = predT.reshape(b, c, _ROWS, _LANES)
    out = _run(predT)
    return out[:, :_MAX_DET, :6]
